# Initial kernel scaffold; baseline (speedup 1.0000x reference)
import jax
import jax.numpy as jnp
from jax.experimental import pallas as pl


def kernel(x, edge_index, W1, att_src1, att_dst1, b1, W2, att_src2, att_dst2, b2):
    raise NotImplementedError("write your pallas kernel here")


# speedup vs baseline: 22.9547x; 22.9547x over previous
import functools

import jax
import jax.numpy as jnp
from jax import lax
from jax.experimental import pallas as pl
from jax.experimental.pallas import tpu as pltpu

N = 2000
E_EDGES = 32000
H1, F1 = 8, 16
D_HID = H1 * F1
DB = 400
NEG = -1e30


def _layer1_pre_body(x_ref, w1_ref, eat_ref, h_ref, a_ref):
    h = jnp.dot(x_ref[...], w1_ref[...], preferred_element_type=jnp.float32)
    h_ref[...] = h
    a_ref[...] = jnp.dot(h, eat_ref[...], preferred_element_type=jnp.float32)


def _attn1_body(c_ref, h_ref, asr_ref, adc_ref, b1_ref, out_ref):
    c = c_ref[...]
    mask = c > 0.0
    h = h_ref[...]
    for hh in range(H1):
        a_dst = adc_ref[:, hh : hh + 1]
        a_src = asr_ref[hh : hh + 1, :]
        e = a_dst + a_src
        e = jnp.where(e >= 0.0, e, 0.2 * e)
        e = jnp.where(mask, e, NEG)
        m = jnp.max(e, axis=1, keepdims=True)
        p = c * jnp.exp(e - m)
        denom = jnp.sum(p, axis=1, keepdims=True) + 1e-16
        o = jnp.dot(p, h[:, hh * F1 : (hh + 1) * F1],
                    preferred_element_type=jnp.float32)
        o = o / denom + b1_ref[:, hh * F1 : (hh + 1) * F1]
        out_ref[:, hh * F1 : (hh + 1) * F1] = jnp.where(
            o > 0.0, o, jnp.exp(jnp.minimum(o, 0.0)) - 1.0)


def _layer2_pre_body(h1_ref, w2_ref, att_ref, h2_ref, a2_ref):
    h2 = jnp.dot(h1_ref[...], w2_ref[...], preferred_element_type=jnp.float32)
    h2_ref[...] = h2
    att = att_ref[...]
    a2_ref[...] = jnp.concatenate(
        [jnp.sum(h2 * att[0:1, :], axis=1, keepdims=True),
         jnp.sum(h2 * att[1:2, :], axis=1, keepdims=True)], axis=1)


def _attn2_body(c_ref, h2_ref, asr_ref, adc_ref, b2_ref, out_ref):
    c = c_ref[...]
    e = adc_ref[...] + asr_ref[...]
    e = jnp.where(e >= 0.0, e, 0.2 * e)
    e = jnp.where(c > 0.0, e, NEG)
    m = jnp.max(e, axis=1, keepdims=True)
    p = c * jnp.exp(e - m)
    denom = jnp.sum(p, axis=1, keepdims=True) + 1e-16
    o = jnp.dot(p, h2_ref[...], preferred_element_type=jnp.float32)
    z = o / denom + b2_ref[...]
    zm = z - jnp.max(z, axis=1, keepdims=True)
    out_ref[...] = zm - jnp.log(jnp.sum(jnp.exp(zm), axis=1, keepdims=True))


def _build_counts(edge_index):
    src = edge_index[0]
    dst = edge_index[1]
    c = jnp.zeros((N, N), jnp.float32).at[dst, src].add(1.0)
    return c + jnp.eye(N, dtype=jnp.float32)


def kernel(x, edge_index, W1, att_src1, att_dst1, b1, W2, att_src2, att_dst2, b2):
    f32 = jnp.float32

    c = _build_counts(edge_index)

    eye = jnp.eye(H1, dtype=f32)
    ea_src = (eye[:, :, None] * att_src1[:, None, :]).reshape(H1, D_HID)
    ea_dst = (eye[:, :, None] * att_dst1[:, None, :]).reshape(H1, D_HID)
    eat = jnp.concatenate([ea_src.T, ea_dst.T], axis=1)

    h1, a1 = pl.pallas_call(
        _layer1_pre_body,
        out_shape=[jax.ShapeDtypeStruct((N, D_HID), f32),
                   jax.ShapeDtypeStruct((N, 2 * H1), f32)],
    )(x, W1, eat)
    asr1 = a1[:, :H1].T
    adc1 = a1[:, H1:]

    grid1 = (N // DB,)
    h1a = pl.pallas_call(
        _attn1_body,
        grid=grid1,
        in_specs=[
            pl.BlockSpec((DB, N), lambda i: (i, 0)),
            pl.BlockSpec((N, D_HID), lambda i: (0, 0)),
            pl.BlockSpec((H1, N), lambda i: (0, 0)),
            pl.BlockSpec((DB, H1), lambda i: (i, 0)),
            pl.BlockSpec((1, D_HID), lambda i: (0, 0)),
        ],
        out_specs=pl.BlockSpec((DB, D_HID), lambda i: (i, 0)),
        out_shape=jax.ShapeDtypeStruct((N, D_HID), f32),
    )(c, h1, asr1, adc1, b1.reshape(1, D_HID))

    att2 = jnp.concatenate([att_src2, att_dst2], axis=0)
    h2, a2 = pl.pallas_call(
        _layer2_pre_body,
        out_shape=[jax.ShapeDtypeStruct((N, N), f32),
                   jax.ShapeDtypeStruct((N, 2), f32)],
    )(h1a, W2, att2)
    asr2 = a2[:, 0:1].T
    adc2 = a2[:, 1:2]

    out = pl.pallas_call(
        _attn2_body,
        grid=grid1,
        in_specs=[
            pl.BlockSpec((DB, N), lambda i: (i, 0)),
            pl.BlockSpec((N, N), lambda i: (0, 0)),
            pl.BlockSpec((1, N), lambda i: (0, 0)),
            pl.BlockSpec((DB, 1), lambda i: (i, 0)),
            pl.BlockSpec((1, N), lambda i: (0, 0)),
        ],
        out_specs=pl.BlockSpec((DB, N), lambda i: (i, 0)),
        out_shape=jax.ShapeDtypeStruct((N, N), f32),
    )(c, h2, asr2, adc2, b2.reshape(1, N))
    return out


# speedup vs baseline: 25.0166x; 1.0898x over previous
import functools

import jax
import jax.numpy as jnp
from jax import lax
from jax.experimental import pallas as pl
from jax.experimental.pallas import tpu as pltpu
from jax.experimental.pallas import tpu_sc as plsc

N = 2000
E_EDGES = 32000
H1, F1 = 8, 16
D_HID = H1 * F1
DB = 400
NEG = -1e30


def _layer1_pre_body(x_ref, w1_ref, eat_ref, h_ref, a_ref):
    h = jnp.dot(x_ref[...], w1_ref[...], preferred_element_type=jnp.float32)
    h_ref[...] = h
    a_ref[...] = jnp.dot(h, eat_ref[...], preferred_element_type=jnp.float32)


def _attn1_body(c_ref, h_ref, asr_ref, adc_ref, b1_ref, out_ref):
    c = c_ref[...]
    mask = c > 0.0
    h = h_ref[...]
    for hh in range(H1):
        a_dst = adc_ref[:, hh : hh + 1]
        a_src = asr_ref[hh : hh + 1, :]
        e = a_dst + a_src
        e = jnp.where(e >= 0.0, e, 0.2 * e)
        e = jnp.where(mask, e, NEG)
        m = jnp.max(e, axis=1, keepdims=True)
        p = c * jnp.exp(e - m)
        denom = jnp.sum(p, axis=1, keepdims=True) + 1e-16
        o = jnp.dot(p, h[:, hh * F1 : (hh + 1) * F1],
                    preferred_element_type=jnp.float32)
        o = o / denom + b1_ref[:, hh * F1 : (hh + 1) * F1]
        out_ref[:, hh * F1 : (hh + 1) * F1] = jnp.where(
            o > 0.0, o, jnp.exp(jnp.minimum(o, 0.0)) - 1.0)


def _layer2_pre_body(h1_ref, w2_ref, att_ref, h2_ref, a2_ref):
    h2 = jnp.dot(h1_ref[...], w2_ref[...], preferred_element_type=jnp.float32)
    h2_ref[...] = h2
    att = att_ref[...]
    a2_ref[...] = jnp.concatenate(
        [jnp.sum(h2 * att[0:1, :], axis=1, keepdims=True),
         jnp.sum(h2 * att[1:2, :], axis=1, keepdims=True)], axis=1)


def _attn2_body(c_ref, h2_ref, asr_ref, adc_ref, b2_ref, out_ref):
    c = c_ref[...]
    e = adc_ref[...] + asr_ref[...]
    e = jnp.where(e >= 0.0, e, 0.2 * e)
    e = jnp.where(c > 0.0, e, NEG)
    m = jnp.max(e, axis=1, keepdims=True)
    p = c * jnp.exp(e - m)
    denom = jnp.sum(p, axis=1, keepdims=True) + 1e-16
    o = jnp.dot(p, h2_ref[...], preferred_element_type=jnp.float32)
    z = o / denom + b2_ref[...]
    zm = z - jnp.max(z, axis=1, keepdims=True)
    out_ref[...] = zm - jnp.log(jnp.sum(jnp.exp(zm), axis=1, keepdims=True))


_SC_NC, _SC_NS = 2, 16
_ROWS = 64
_NPAD = _SC_NC * _SC_NS * _ROWS
_CHUNK = 800
_N_CHUNKS = E_EDGES // _CHUNK


def _counts_sc_body(src_hbm, dst_hbm, c_hbm, srcb, dstb, acc):
    wid = lax.axis_index("s") * _SC_NC + lax.axis_index("c")
    base = wid * _ROWS
    zeros16 = jnp.zeros((16,), jnp.float32)
    ones16 = jnp.ones((16,), jnp.float32)
    lanes = lax.iota(jnp.int32, 16)

    def zbody(i, carry):
        for k in range(8):
            acc[pl.ds((i * 8 + k) * 16, 16)] = zeros16
        return carry
    lax.fori_loop(0, _ROWS * N // (16 * 8), zbody, 0)

    for g in range(4):
        ln = lanes + g * 16
        col = base + ln
        plsc.addupdate_scatter(acc, [ln * N + col], ones16, mask=col < N)

    def chunk_body(c, carry):
        pltpu.sync_copy(src_hbm.at[pl.ds(c * _CHUNK, _CHUNK)], srcb)
        pltpu.sync_copy(dst_hbm.at[pl.ds(c * _CHUNK, _CHUNK)], dstb)

        def vbody(i, inner):
            sv = srcb[pl.ds(i * 16, 16)]
            dv = dstb[pl.ds(i * 16, 16)]
            loc = dv - base
            m = (loc >= 0) & (loc < _ROWS)
            plsc.addupdate_scatter(acc, [loc * N + sv], ones16, mask=m)
            return inner
        lax.fori_loop(0, _CHUNK // 16, vbody, 0)
        return carry
    lax.fori_loop(0, _N_CHUNKS, chunk_body, 0)

    pltpu.sync_copy(acc, c_hbm.at[pl.ds(base * N, _ROWS * N)])


def _build_counts(edge_index):
    src = edge_index[0]
    dst = edge_index[1]
    c_flat = pl.kernel(
        _counts_sc_body,
        out_type=jax.ShapeDtypeStruct((_NPAD * N,), jnp.float32),
        mesh=plsc.VectorSubcoreMesh(
            core_axis_name="c", subcore_axis_name="s",
            num_cores=_SC_NC, num_subcores=_SC_NS),
        compiler_params=pltpu.CompilerParams(needs_layout_passes=False),
        scratch_types=[
            pltpu.VMEM((_CHUNK,), jnp.int32),
            pltpu.VMEM((_CHUNK,), jnp.int32),
            pltpu.VMEM((_ROWS * N,), jnp.float32),
        ],
    )(src, dst)
    return c_flat.reshape(_NPAD, N)[:N]


def kernel(x, edge_index, W1, att_src1, att_dst1, b1, W2, att_src2, att_dst2, b2):
    f32 = jnp.float32

    c = _build_counts(edge_index)

    eye = jnp.eye(H1, dtype=f32)
    ea_src = (eye[:, :, None] * att_src1[:, None, :]).reshape(H1, D_HID)
    ea_dst = (eye[:, :, None] * att_dst1[:, None, :]).reshape(H1, D_HID)
    eat = jnp.concatenate([ea_src.T, ea_dst.T], axis=1)

    h1, a1 = pl.pallas_call(
        _layer1_pre_body,
        out_shape=[jax.ShapeDtypeStruct((N, D_HID), f32),
                   jax.ShapeDtypeStruct((N, 2 * H1), f32)],
    )(x, W1, eat)
    asr1 = a1[:, :H1].T
    adc1 = a1[:, H1:]

    grid1 = (N // DB,)
    h1a = pl.pallas_call(
        _attn1_body,
        grid=grid1,
        in_specs=[
            pl.BlockSpec((DB, N), lambda i: (i, 0)),
            pl.BlockSpec((N, D_HID), lambda i: (0, 0)),
            pl.BlockSpec((H1, N), lambda i: (0, 0)),
            pl.BlockSpec((DB, H1), lambda i: (i, 0)),
            pl.BlockSpec((1, D_HID), lambda i: (0, 0)),
        ],
        out_specs=pl.BlockSpec((DB, D_HID), lambda i: (i, 0)),
        out_shape=jax.ShapeDtypeStruct((N, D_HID), f32),
    )(c, h1, asr1, adc1, b1.reshape(1, D_HID))

    att2 = jnp.concatenate([att_src2, att_dst2], axis=0)
    h2, a2 = pl.pallas_call(
        _layer2_pre_body,
        out_shape=[jax.ShapeDtypeStruct((N, N), f32),
                   jax.ShapeDtypeStruct((N, 2), f32)],
    )(h1a, W2, att2)
    asr2 = a2[:, 0:1].T
    adc2 = a2[:, 1:2]

    out = pl.pallas_call(
        _attn2_body,
        grid=grid1,
        in_specs=[
            pl.BlockSpec((DB, N), lambda i: (i, 0)),
            pl.BlockSpec((N, N), lambda i: (0, 0)),
            pl.BlockSpec((1, N), lambda i: (0, 0)),
            pl.BlockSpec((DB, 1), lambda i: (i, 0)),
            pl.BlockSpec((1, N), lambda i: (0, 0)),
        ],
        out_specs=pl.BlockSpec((DB, N), lambda i: (i, 0)),
        out_shape=jax.ShapeDtypeStruct((N, N), f32),
    )(c, h2, asr2, adc2, b2.reshape(1, N))
    return out


# speedup vs baseline: 32.5851x; 1.3025x over previous
import functools

import jax
import jax.numpy as jnp
from jax import lax
from jax.experimental import pallas as pl
from jax.experimental.pallas import tpu as pltpu
from jax.experimental.pallas import tpu_sc as plsc

N = 2000
E_EDGES = 32000
H1, F1 = 8, 16
D_HID = H1 * F1
DB = 400
NEG = -1e30


def _layer1_pre_body(x_ref, w1_ref, eat_ref, w2_ref, att2_ref,
                     h_ref, a_ref, w2att_ref):
    h = jnp.dot(x_ref[...], w1_ref[...], preferred_element_type=jnp.float32)
    h_ref[...] = h
    a_ref[...] = jnp.dot(h, eat_ref[...], preferred_element_type=jnp.float32)
    w2att_ref[...] = lax.dot_general(
        w2_ref[...], att2_ref[...], (((1,), (1,)), ((), ())),
        preferred_element_type=jnp.float32)


def _attn1_body(c_ref, h_ref, asr_ref, adc_ref, b1_ref, w2att_ref,
                out_ref, a2_ref):
    c = c_ref[...]
    nb = c.shape[0]
    haug = jnp.concatenate(
        [h_ref[...], jnp.ones((N, 1), jnp.float32)], axis=1)
    for hh in range(H1):
        a_s = asr_ref[hh : hh + 1, :]
        a_d = adc_ref[:, hh : hh + 1]
        ag = jnp.max(a_s, axis=1, keepdims=True)
        t = a_d + ag
        bnd = jnp.where(t >= 0.0, t, 0.2 * t)
        u = jnp.exp(a_d - bnd)
        up = jnp.exp(0.2 * a_d - bnd)
        v = jnp.exp(a_s)
        vp = jnp.exp(0.2 * a_s)
        p = c * jnp.maximum(u * v, up * vp)
        o_aug = jnp.dot(
            p, jnp.concatenate(
                [haug[:, hh * F1 : (hh + 1) * F1], haug[:, D_HID:]], axis=1),
            preferred_element_type=jnp.float32)
        o = (o_aug[:, :F1] / (o_aug[:, F1 : F1 + 1] + 1e-16)
             + b1_ref[:, hh * F1 : (hh + 1) * F1])
        out_ref[:, hh * F1 : (hh + 1) * F1] = jnp.where(
            o > 0.0, o, jnp.exp(jnp.minimum(o, 0.0)) - 1.0)
    a2_ref[...] = jnp.dot(out_ref[...], w2att_ref[...],
                          preferred_element_type=jnp.float32)


def _attn2_body(c_ref, h1a_ref, w2_ref, asr_ref, adc_ref, b2_ref, out_ref):
    c = c_ref[...]
    a_s = asr_ref[...]
    a_d = adc_ref[...]
    ag = jnp.max(a_s, axis=1, keepdims=True)
    t = a_d + ag
    bnd = jnp.where(t >= 0.0, t, 0.2 * t)
    u = jnp.exp(a_d - bnd)
    up = jnp.exp(0.2 * a_d - bnd)
    v = jnp.exp(a_s)
    vp = jnp.exp(0.2 * a_s)
    p = c * jnp.maximum(u * v, up * vp)
    haug = jnp.concatenate(
        [h1a_ref[...], jnp.ones((N, 1), jnp.float32)], axis=1)
    a_aug = jnp.dot(p, haug, preferred_element_type=jnp.float32)
    a = a_aug[:, :D_HID] / (a_aug[:, D_HID : D_HID + 1] + 1e-16)
    z = jnp.dot(a, w2_ref[...], preferred_element_type=jnp.float32) + b2_ref[...]
    zm = z - jnp.max(z, axis=1, keepdims=True)
    out_ref[...] = zm - jnp.log(jnp.sum(jnp.exp(zm), axis=1, keepdims=True))


_SC_NC, _SC_NS = 2, 16
_ROWS = 64
_NPAD = _SC_NC * _SC_NS * _ROWS
_CHUNK = 800
_N_CHUNKS = E_EDGES // _CHUNK


def _counts_sc_body(src_hbm, dst_hbm, c_hbm, srcb, dstb, acc):
    wid = lax.axis_index("s") * _SC_NC + lax.axis_index("c")
    base = wid * _ROWS
    zeros16 = jnp.zeros((16,), jnp.float32)
    ones16 = jnp.ones((16,), jnp.float32)
    lanes = lax.iota(jnp.int32, 16)

    def zbody(i, carry):
        for k in range(8):
            acc[pl.ds((i * 8 + k) * 16, 16)] = zeros16
        return carry
    lax.fori_loop(0, _ROWS * N // (16 * 8), zbody, 0)

    for g in range(4):
        ln = lanes + g * 16
        col = base + ln
        plsc.addupdate_scatter(acc, [ln * N + col], ones16, mask=col < N)

    def chunk_body(c, carry):
        pltpu.sync_copy(src_hbm.at[pl.ds(c * _CHUNK, _CHUNK)], srcb)
        pltpu.sync_copy(dst_hbm.at[pl.ds(c * _CHUNK, _CHUNK)], dstb)

        def vbody(i, inner):
            sv = srcb[pl.ds(i * 16, 16)]
            dv = dstb[pl.ds(i * 16, 16)]
            loc = dv - base
            m = (loc >= 0) & (loc < _ROWS)
            plsc.addupdate_scatter(acc, [loc * N + sv], ones16, mask=m)
            return inner
        lax.fori_loop(0, _CHUNK // 16, vbody, 0)
        return carry
    lax.fori_loop(0, _N_CHUNKS, chunk_body, 0)

    pltpu.sync_copy(acc, c_hbm.at[pl.ds(base * N, _ROWS * N)])


def _build_counts(edge_index):
    src = edge_index[0]
    dst = edge_index[1]
    c_flat = pl.kernel(
        _counts_sc_body,
        out_type=jax.ShapeDtypeStruct((_NPAD * N,), jnp.float32),
        mesh=plsc.VectorSubcoreMesh(
            core_axis_name="c", subcore_axis_name="s",
            num_cores=_SC_NC, num_subcores=_SC_NS),
        compiler_params=pltpu.CompilerParams(needs_layout_passes=False),
        scratch_types=[
            pltpu.VMEM((_CHUNK,), jnp.int32),
            pltpu.VMEM((_CHUNK,), jnp.int32),
            pltpu.VMEM((_ROWS * N,), jnp.float32),
        ],
    )(src, dst)
    return c_flat.reshape(_NPAD, N)[:N]


def kernel(x, edge_index, W1, att_src1, att_dst1, b1, W2, att_src2, att_dst2, b2):
    f32 = jnp.float32

    c = _build_counts(edge_index)

    eye = jnp.eye(H1, dtype=f32)
    ea_src = (eye[:, :, None] * att_src1[:, None, :]).reshape(H1, D_HID)
    ea_dst = (eye[:, :, None] * att_dst1[:, None, :]).reshape(H1, D_HID)
    eat = jnp.concatenate([ea_src.T, ea_dst.T], axis=1)

    att2 = jnp.concatenate([att_src2, att_dst2], axis=0)
    h1, a1, w2att = pl.pallas_call(
        _layer1_pre_body,
        out_shape=[jax.ShapeDtypeStruct((N, D_HID), f32),
                   jax.ShapeDtypeStruct((N, 2 * H1), f32),
                   jax.ShapeDtypeStruct((D_HID, 2), f32)],
    )(x, W1, eat, W2, att2)
    asr1 = a1[:, :H1].T
    adc1 = a1[:, H1:]

    grid1 = (N // DB,)
    h1a, a2 = pl.pallas_call(
        _attn1_body,
        grid=grid1,
        in_specs=[
            pl.BlockSpec((DB, N), lambda i: (i, 0)),
            pl.BlockSpec((N, D_HID), lambda i: (0, 0)),
            pl.BlockSpec((H1, N), lambda i: (0, 0)),
            pl.BlockSpec((DB, H1), lambda i: (i, 0)),
            pl.BlockSpec((1, D_HID), lambda i: (0, 0)),
            pl.BlockSpec((D_HID, 2), lambda i: (0, 0)),
        ],
        out_specs=[pl.BlockSpec((DB, D_HID), lambda i: (i, 0)),
                   pl.BlockSpec((DB, 2), lambda i: (i, 0))],
        out_shape=[jax.ShapeDtypeStruct((N, D_HID), f32),
                   jax.ShapeDtypeStruct((N, 2), f32)],
    )(c, h1, asr1, adc1, b1.reshape(1, D_HID), w2att)
    asr2 = a2[:, 0:1].T
    adc2 = a2[:, 1:2]

    out = pl.pallas_call(
        _attn2_body,
        grid=grid1,
        in_specs=[
            pl.BlockSpec((DB, N), lambda i: (i, 0)),
            pl.BlockSpec((N, D_HID), lambda i: (0, 0)),
            pl.BlockSpec((D_HID, N), lambda i: (0, 0)),
            pl.BlockSpec((1, N), lambda i: (0, 0)),
            pl.BlockSpec((DB, 1), lambda i: (i, 0)),
            pl.BlockSpec((1, N), lambda i: (0, 0)),
        ],
        out_specs=pl.BlockSpec((DB, N), lambda i: (i, 0)),
        out_shape=jax.ShapeDtypeStruct((N, N), f32),
    )(c, h1a, W2, asr2, adc2, b2.reshape(1, N))
    return out


# speedup vs baseline: 41.3861x; 1.2701x over previous
import functools

import jax
import jax.numpy as jnp
from jax import lax
from jax.experimental import pallas as pl
from jax.experimental.pallas import tpu as pltpu
from jax.experimental.pallas import tpu_sc as plsc

N = 2000
E_EDGES = 32000
H1, F1 = 8, 16
D_HID = H1 * F1
DB = 400
NEG = -1e30


def _layer1_pre_body(x_ref, w1_ref, eat_ref, w2_ref, att2_ref,
                     h_ref, a_ref, w2att_ref):
    h = jnp.dot(x_ref[...], w1_ref[...], preferred_element_type=jnp.float32)
    h_ref[...] = h
    a_ref[...] = jnp.dot(h, eat_ref[...], preferred_element_type=jnp.float32)
    w2att_ref[...] = lax.dot_general(
        w2_ref[...], att2_ref[...], (((1,), (1,)), ((), ())),
        preferred_element_type=jnp.float32)


def _attn1_body(c_ref, h_ref, asr_ref, adc_ref, b1_ref, w2att_ref,
                out_ref, a2_ref):
    c = c_ref[...]
    nb = c.shape[0]
    haug = jnp.concatenate(
        [h_ref[...], jnp.ones((N, 1), jnp.float32)], axis=1)
    for hh in range(H1):
        a_s = asr_ref[hh : hh + 1, :]
        a_d = adc_ref[:, hh : hh + 1]
        ag = jnp.max(a_s, axis=1, keepdims=True)
        t = a_d + ag
        bnd = jnp.where(t >= 0.0, t, 0.2 * t)
        u = jnp.exp(a_d - bnd)
        up = jnp.exp(0.2 * a_d - bnd)
        v = jnp.exp(a_s)
        vp = jnp.exp(0.2 * a_s)
        p = c * jnp.maximum(u * v, up * vp)
        o_aug = jnp.dot(
            p, jnp.concatenate(
                [haug[:, hh * F1 : (hh + 1) * F1], haug[:, D_HID:]], axis=1),
            preferred_element_type=jnp.float32)
        o = (o_aug[:, :F1] / (o_aug[:, F1 : F1 + 1] + 1e-16)
             + b1_ref[:, hh * F1 : (hh + 1) * F1])
        out_ref[:, hh * F1 : (hh + 1) * F1] = jnp.where(
            o > 0.0, o, jnp.exp(jnp.minimum(o, 0.0)) - 1.0)
    a2_ref[...] = jnp.dot(out_ref[...], w2att_ref[...],
                          preferred_element_type=jnp.float32)


def _attn2_body(c_ref, h1a_ref, w2_ref, asr_ref, adc_ref, b2_ref, out_ref):
    c = c_ref[...]
    a_s = asr_ref[...]
    a_d = adc_ref[...]
    ag = jnp.max(a_s, axis=1, keepdims=True)
    t = a_d + ag
    bnd = jnp.where(t >= 0.0, t, 0.2 * t)
    u = jnp.exp(a_d - bnd)
    up = jnp.exp(0.2 * a_d - bnd)
    v = jnp.exp(a_s)
    vp = jnp.exp(0.2 * a_s)
    p = c * jnp.maximum(u * v, up * vp)
    haug = jnp.concatenate(
        [h1a_ref[...], jnp.ones((N, 1), jnp.float32)], axis=1)
    a_aug = jnp.dot(p, haug, preferred_element_type=jnp.float32)
    a = a_aug[:, :D_HID] / (a_aug[:, D_HID : D_HID + 1] + 1e-16)
    z = jnp.dot(a, w2_ref[...], preferred_element_type=jnp.float32) + b2_ref[...]
    zm = z - jnp.max(z, axis=1, keepdims=True)
    out_ref[...] = zm - jnp.log(jnp.sum(jnp.exp(zm), axis=1, keepdims=True))


_SC_NC, _SC_NS = 2, 16
_ROWS = 64
_NPAD = _SC_NC * _SC_NS * _ROWS
_CHUNK = 320
_N_CHUNKS = E_EDGES // _CHUNK
_NBUF = 4


def _counts_sc_body(ed_hbm, c_hbm, b0, b1, b2, b3, s0, s1, s2, s3, acc):
    bufs = (b0, b1, b2, b3)
    sems = (s0, s1, s2, s3)
    wid = lax.axis_index("s") * _SC_NC + lax.axis_index("c")
    base = wid * _ROWS
    zeros16 = jnp.zeros((16,), jnp.float32)
    ones16 = jnp.ones((16,), jnp.float32)
    lanes = lax.iota(jnp.int32, 16)

    handles = {}
    for c in range(_NBUF):
        handles[c] = pltpu.async_copy(
            ed_hbm.at[pl.ds(c * 2 * _CHUNK, 2 * _CHUNK)], bufs[c], sems[c])

    def zbody(i, carry):
        for k in range(8):
            acc[pl.ds((i * 8 + k) * 16, 16)] = zeros16
        return carry
    lax.fori_loop(0, _ROWS * N // (16 * 8), zbody, 0)

    for g in range(4):
        ln = lanes + g * 16
        col = base + ln
        plsc.addupdate_scatter(acc, [ln * N + col], ones16, mask=col < N)

    for c in range(_N_CHUNKS):
        b = bufs[c % _NBUF]
        handles[c].wait()
        if c + _NBUF < _N_CHUNKS:
            handles[c + _NBUF] = pltpu.async_copy(
                ed_hbm.at[pl.ds((c + _NBUF) * 2 * _CHUNK, 2 * _CHUNK)],
                bufs[(c + _NBUF) % _NBUF], sems[(c + _NBUF) % _NBUF])

        def vbody(i, inner):
            sv = b[pl.ds(i * 16, 16)]
            dv = b[pl.ds(_CHUNK + i * 16, 16)]
            loc = dv - base
            m = (loc >= 0) & (loc < _ROWS)
            plsc.addupdate_scatter(acc, [loc * N + sv], ones16, mask=m)
            return inner
        lax.fori_loop(0, _CHUNK // 16, vbody, 0)

    pltpu.sync_copy(acc, c_hbm.at[pl.ds(base * N, _ROWS * N)])


def _build_counts(edge_index):
    ed = jnp.concatenate(
        [edge_index[0].reshape(_N_CHUNKS, _CHUNK),
         edge_index[1].reshape(_N_CHUNKS, _CHUNK)], axis=1).reshape(-1)
    c_flat = pl.kernel(
        _counts_sc_body,
        out_type=jax.ShapeDtypeStruct((_NPAD * N,), jnp.float32),
        mesh=plsc.VectorSubcoreMesh(
            core_axis_name="c", subcore_axis_name="s",
            num_cores=_SC_NC, num_subcores=_SC_NS),
        compiler_params=pltpu.CompilerParams(needs_layout_passes=False),
        scratch_types=(
            [pltpu.VMEM((2 * _CHUNK,), jnp.int32)] * _NBUF
            + [pltpu.SemaphoreType.DMA] * _NBUF
            + [pltpu.VMEM((_ROWS * N,), jnp.float32)]
        ),
    )(ed)
    return c_flat.reshape(_NPAD, N)[:N]


def kernel(x, edge_index, W1, att_src1, att_dst1, b1, W2, att_src2, att_dst2, b2):
    f32 = jnp.float32

    c = _build_counts(edge_index)

    eye = jnp.eye(H1, dtype=f32)
    ea_src = (eye[:, :, None] * att_src1[:, None, :]).reshape(H1, D_HID)
    ea_dst = (eye[:, :, None] * att_dst1[:, None, :]).reshape(H1, D_HID)
    eat = jnp.concatenate([ea_src.T, ea_dst.T], axis=1)

    att2 = jnp.concatenate([att_src2, att_dst2], axis=0)
    h1, a1, w2att = pl.pallas_call(
        _layer1_pre_body,
        out_shape=[jax.ShapeDtypeStruct((N, D_HID), f32),
                   jax.ShapeDtypeStruct((N, 2 * H1), f32),
                   jax.ShapeDtypeStruct((D_HID, 2), f32)],
    )(x, W1, eat, W2, att2)
    asr1 = a1[:, :H1].T
    adc1 = a1[:, H1:]

    grid1 = (N // DB,)
    h1a, a2 = pl.pallas_call(
        _attn1_body,
        grid=grid1,
        in_specs=[
            pl.BlockSpec((DB, N), lambda i: (i, 0)),
            pl.BlockSpec((N, D_HID), lambda i: (0, 0)),
            pl.BlockSpec((H1, N), lambda i: (0, 0)),
            pl.BlockSpec((DB, H1), lambda i: (i, 0)),
            pl.BlockSpec((1, D_HID), lambda i: (0, 0)),
            pl.BlockSpec((D_HID, 2), lambda i: (0, 0)),
        ],
        out_specs=[pl.BlockSpec((DB, D_HID), lambda i: (i, 0)),
                   pl.BlockSpec((DB, 2), lambda i: (i, 0))],
        out_shape=[jax.ShapeDtypeStruct((N, D_HID), f32),
                   jax.ShapeDtypeStruct((N, 2), f32)],
    )(c, h1, asr1, adc1, b1.reshape(1, D_HID), w2att)
    asr2 = a2[:, 0:1].T
    adc2 = a2[:, 1:2]

    out = pl.pallas_call(
        _attn2_body,
        grid=grid1,
        in_specs=[
            pl.BlockSpec((DB, N), lambda i: (i, 0)),
            pl.BlockSpec((N, D_HID), lambda i: (0, 0)),
            pl.BlockSpec((D_HID, N), lambda i: (0, 0)),
            pl.BlockSpec((1, N), lambda i: (0, 0)),
            pl.BlockSpec((DB, 1), lambda i: (i, 0)),
            pl.BlockSpec((1, N), lambda i: (0, 0)),
        ],
        out_specs=pl.BlockSpec((DB, N), lambda i: (i, 0)),
        out_shape=jax.ShapeDtypeStruct((N, N), f32),
    )(c, h1a, W2, asr2, adc2, b2.reshape(1, N))
    return out


# speedup vs baseline: 45.6286x; 1.1025x over previous
import functools

import jax
import jax.numpy as jnp
from jax import lax
from jax.experimental import pallas as pl
from jax.experimental.pallas import tpu as pltpu
from jax.experimental.pallas import tpu_sc as plsc

N = 2000
E_EDGES = 32000
H1, F1 = 8, 16
D_HID = H1 * F1
DB = 400
NEG = -1e30


def _layer1_pre_body(x_ref, w1_ref, h_ref):
    h_ref[...] = jnp.dot(x_ref[...], w1_ref[...],
                         preferred_element_type=jnp.float32)


def _head_proj(att_ref):
    att = att_ref[...]
    tiled = jnp.concatenate([att] * H1, axis=1)
    row = lax.broadcasted_iota(jnp.int32, (H1, D_HID), 0)
    col = lax.broadcasted_iota(jnp.int32, (H1, D_HID), 1)
    return jnp.where(col // F1 == row, tiled, 0.0)


def _attn1_body(c_ref, h_ref, hblk_ref, asrc_ref, adst_ref, b1_ref, out_ref):
    c = c_ref[...]
    h1 = h_ref[...]
    asr = lax.dot_general(_head_proj(asrc_ref), h1, (((1,), (1,)), ((), ())),
                          preferred_element_type=jnp.float32)
    adc = lax.dot_general(hblk_ref[...], _head_proj(adst_ref),
                          (((1,), (1,)), ((), ())),
                          preferred_element_type=jnp.float32)
    haug = jnp.concatenate(
        [h1, jnp.ones((N, 1), jnp.float32)], axis=1)
    for hh in range(H1):
        a_s = asr[hh : hh + 1, :]
        a_d = adc[:, hh : hh + 1]
        ag = jnp.max(a_s, axis=1, keepdims=True)
        t = a_d + ag
        bnd = jnp.where(t >= 0.0, t, 0.2 * t)
        u = jnp.exp(a_d - bnd)
        up = jnp.exp(0.2 * a_d - bnd)
        v = jnp.exp(a_s)
        vp = jnp.exp(0.2 * a_s)
        p = c * jnp.maximum(u * v, up * vp)
        o_aug = jnp.dot(
            p, jnp.concatenate(
                [haug[:, hh * F1 : (hh + 1) * F1], haug[:, D_HID:]], axis=1),
            preferred_element_type=jnp.float32)
        o = (o_aug[:, :F1] / (o_aug[:, F1 : F1 + 1] + 1e-16)
             + b1_ref[:, hh * F1 : (hh + 1) * F1])
        out_ref[:, hh * F1 : (hh + 1) * F1] = jnp.where(
            o > 0.0, o, jnp.exp(jnp.minimum(o, 0.0)) - 1.0)


def _attn2_body(c_ref, h1a_ref, hblk_ref, w2_ref, att2_ref, b2_ref, out_ref):
    c = c_ref[...]
    h1a = h1a_ref[...]
    w2att = lax.dot_general(w2_ref[...], att2_ref[...],
                            (((1,), (1,)), ((), ())),
                            preferred_element_type=jnp.float32)
    a_s = lax.dot_general(w2att[:, 0:1], h1a, (((0,), (1,)), ((), ())),
                          preferred_element_type=jnp.float32)
    a_d = jnp.dot(hblk_ref[...], w2att[:, 1:2],
                  preferred_element_type=jnp.float32)
    ag = jnp.max(a_s, axis=1, keepdims=True)
    t = a_d + ag
    bnd = jnp.where(t >= 0.0, t, 0.2 * t)
    u = jnp.exp(a_d - bnd)
    up = jnp.exp(0.2 * a_d - bnd)
    v = jnp.exp(a_s)
    vp = jnp.exp(0.2 * a_s)
    p = c * jnp.maximum(u * v, up * vp)
    haug = jnp.concatenate(
        [h1a, jnp.ones((N, 1), jnp.float32)], axis=1)
    a_aug = jnp.dot(p, haug, preferred_element_type=jnp.float32)
    a = a_aug[:, :D_HID] / (a_aug[:, D_HID : D_HID + 1] + 1e-16)
    z = jnp.dot(a, w2_ref[...], preferred_element_type=jnp.float32) + b2_ref[...]
    zm = z - jnp.max(z, axis=1, keepdims=True)
    out_ref[...] = zm - jnp.log(jnp.sum(jnp.exp(zm), axis=1, keepdims=True))


_SC_NC, _SC_NS = 2, 16
_ROWS = 64
_NPAD = _SC_NC * _SC_NS * _ROWS
_CHUNK = 320
_N_CHUNKS = E_EDGES // _CHUNK
_NBUF = 4


def _counts_sc_body(ed_hbm, c_hbm, b0, b1, b2, b3, s0, s1, s2, s3, acc):
    bufs = (b0, b1, b2, b3)
    sems = (s0, s1, s2, s3)
    wid = lax.axis_index("s") * _SC_NC + lax.axis_index("c")
    base = wid * _ROWS
    zeros16 = jnp.zeros((16,), jnp.float32)
    ones16 = jnp.ones((16,), jnp.float32)
    lanes = lax.iota(jnp.int32, 16)

    handles = {}
    for c in range(_NBUF):
        handles[c] = pltpu.async_copy(
            ed_hbm.at[pl.ds(c * 2 * _CHUNK, 2 * _CHUNK)], bufs[c], sems[c])

    def zbody(i, carry):
        for k in range(8):
            acc[pl.ds((i * 8 + k) * 16, 16)] = zeros16
        return carry
    lax.fori_loop(0, _ROWS * N // (16 * 8), zbody, 0)

    for g in range(4):
        ln = lanes + g * 16
        col = base + ln
        plsc.addupdate_scatter(acc, [ln * N + col], ones16, mask=col < N)

    for c in range(_N_CHUNKS):
        b = bufs[c % _NBUF]
        handles[c].wait()
        if c + _NBUF < _N_CHUNKS:
            handles[c + _NBUF] = pltpu.async_copy(
                ed_hbm.at[pl.ds((c + _NBUF) * 2 * _CHUNK, 2 * _CHUNK)],
                bufs[(c + _NBUF) % _NBUF], sems[(c + _NBUF) % _NBUF])

        def vbody(i, inner):
            sv = b[pl.ds(i * 16, 16)]
            dv = b[pl.ds(_CHUNK + i * 16, 16)]
            loc = dv - base
            m = (loc >= 0) & (loc < _ROWS)
            plsc.addupdate_scatter(acc, [loc * N + sv], ones16, mask=m)
            return inner
        lax.fori_loop(0, _CHUNK // 16, vbody, 0)

    pltpu.sync_copy(acc, c_hbm.at[pl.ds(base * N, _ROWS * N)])


def _build_counts(edge_index):
    ed = jnp.concatenate(
        [edge_index[0].reshape(_N_CHUNKS, _CHUNK),
         edge_index[1].reshape(_N_CHUNKS, _CHUNK)], axis=1).reshape(-1)
    c_flat = pl.kernel(
        _counts_sc_body,
        out_type=jax.ShapeDtypeStruct((_NPAD * N,), jnp.float32),
        mesh=plsc.VectorSubcoreMesh(
            core_axis_name="c", subcore_axis_name="s",
            num_cores=_SC_NC, num_subcores=_SC_NS),
        compiler_params=pltpu.CompilerParams(needs_layout_passes=False),
        scratch_types=(
            [pltpu.VMEM((2 * _CHUNK,), jnp.int32)] * _NBUF
            + [pltpu.SemaphoreType.DMA] * _NBUF
            + [pltpu.VMEM((_ROWS * N,), jnp.float32)]
        ),
    )(ed)
    return c_flat.reshape(_NPAD, N)


def kernel(x, edge_index, W1, att_src1, att_dst1, b1, W2, att_src2, att_dst2, b2):
    f32 = jnp.float32

    c = _build_counts(edge_index)
    att2 = jnp.concatenate([att_src2, att_dst2], axis=0)
    grid1 = (N // DB,)

    h1 = pl.pallas_call(
        _layer1_pre_body,
        grid=grid1,
        in_specs=[
            pl.BlockSpec((DB, N), lambda i: (i, 0)),
            pl.BlockSpec((N, D_HID), lambda i: (0, 0)),
        ],
        out_specs=pl.BlockSpec((DB, D_HID), lambda i: (i, 0)),
        out_shape=jax.ShapeDtypeStruct((N, D_HID), f32),
    )(x, W1)

    h1a = pl.pallas_call(
        _attn1_body,
        grid=grid1,
        in_specs=[
            pl.BlockSpec((DB, N), lambda i: (i, 0)),
            pl.BlockSpec((N, D_HID), lambda i: (0, 0)),
            pl.BlockSpec((DB, D_HID), lambda i: (i, 0)),
            pl.BlockSpec((H1, F1), lambda i: (0, 0)),
            pl.BlockSpec((H1, F1), lambda i: (0, 0)),
            pl.BlockSpec((1, D_HID), lambda i: (0, 0)),
        ],
        out_specs=pl.BlockSpec((DB, D_HID), lambda i: (i, 0)),
        out_shape=jax.ShapeDtypeStruct((N, D_HID), f32),
    )(c, h1, h1, att_src1, att_dst1, b1.reshape(1, D_HID))

    out = pl.pallas_call(
        _attn2_body,
        grid=grid1,
        in_specs=[
            pl.BlockSpec((DB, N), lambda i: (i, 0)),
            pl.BlockSpec((N, D_HID), lambda i: (0, 0)),
            pl.BlockSpec((DB, D_HID), lambda i: (i, 0)),
            pl.BlockSpec((D_HID, N), lambda i: (0, 0)),
            pl.BlockSpec((2, N), lambda i: (0, 0)),
            pl.BlockSpec((1, N), lambda i: (0, 0)),
        ],
        out_specs=pl.BlockSpec((DB, N), lambda i: (i, 0)),
        out_shape=jax.ShapeDtypeStruct((N, N), f32),
    )(c, h1a, h1a, W2, att2, b2.reshape(1, N))
    return out


# speedup vs baseline: 46.1661x; 1.0118x over previous
import functools

import jax
import jax.numpy as jnp
from jax import lax
from jax.experimental import pallas as pl
from jax.experimental.pallas import tpu as pltpu
from jax.experimental.pallas import tpu_sc as plsc

N = 2000
E_EDGES = 32000
H1, F1 = 8, 16
D_HID = H1 * F1
DB = 400
NEG = -1e30


def _layer1_pre_body(x_ref, w1_ref, h_ref):
    h_ref[...] = jnp.dot(x_ref[...], w1_ref[...],
                         preferred_element_type=jnp.float32)


def _head_proj(att_ref):
    att = att_ref[...]
    tiled = jnp.concatenate([att] * H1, axis=1)
    row = lax.broadcasted_iota(jnp.int32, (H1, D_HID), 0)
    col = lax.broadcasted_iota(jnp.int32, (H1, D_HID), 1)
    return jnp.where(col // F1 == row, tiled, 0.0)


def _attn1_body(c_ref, h_ref, hblk_ref, asrc_ref, adst_ref, b1_ref, out_ref):
    c = c_ref[...]
    h1 = h_ref[...]
    asr = lax.dot_general(_head_proj(asrc_ref), h1, (((1,), (1,)), ((), ())),
                          preferred_element_type=jnp.float32)
    adc = lax.dot_general(hblk_ref[...], _head_proj(adst_ref),
                          (((1,), (1,)), ((), ())),
                          preferred_element_type=jnp.float32)
    haug = jnp.concatenate(
        [h1, jnp.ones((N, 1), jnp.float32)], axis=1).astype(jnp.bfloat16)
    for hh in range(H1):
        a_s = asr[hh : hh + 1, :]
        a_d = adc[:, hh : hh + 1]
        ag = jnp.max(a_s, axis=1, keepdims=True)
        t = a_d + ag
        bnd = jnp.where(t >= 0.0, t, 0.2 * t)
        u = jnp.exp(a_d - bnd)
        up = jnp.exp(0.2 * a_d - bnd)
        v = jnp.exp(a_s)
        vp = jnp.exp(0.2 * a_s)
        p = (c * jnp.maximum(u * v, up * vp)).astype(jnp.bfloat16)
        o_aug = jnp.dot(
            p, jnp.concatenate(
                [haug[:, hh * F1 : (hh + 1) * F1], haug[:, D_HID:]], axis=1),
            preferred_element_type=jnp.float32)
        o = (o_aug[:, :F1] / (o_aug[:, F1 : F1 + 1] + 1e-16)
             + b1_ref[:, hh * F1 : (hh + 1) * F1])
        out_ref[:, hh * F1 : (hh + 1) * F1] = jnp.where(
            o > 0.0, o, jnp.exp(jnp.minimum(o, 0.0)) - 1.0)


def _attn2_body(c_ref, h1a_ref, hblk_ref, w2_ref, att2_ref, b2_ref, out_ref):
    c = c_ref[...]
    h1a = h1a_ref[...]
    w2att = lax.dot_general(w2_ref[...], att2_ref[...],
                            (((1,), (1,)), ((), ())),
                            preferred_element_type=jnp.float32)
    a_s = lax.dot_general(w2att[:, 0:1], h1a, (((0,), (1,)), ((), ())),
                          preferred_element_type=jnp.float32)
    a_d = jnp.dot(hblk_ref[...], w2att[:, 1:2],
                  preferred_element_type=jnp.float32)
    ag = jnp.max(a_s, axis=1, keepdims=True)
    t = a_d + ag
    bnd = jnp.where(t >= 0.0, t, 0.2 * t)
    u = jnp.exp(a_d - bnd)
    up = jnp.exp(0.2 * a_d - bnd)
    v = jnp.exp(a_s)
    vp = jnp.exp(0.2 * a_s)
    p = c * jnp.maximum(u * v, up * vp)
    haug = jnp.concatenate(
        [h1a, jnp.ones((N, 1), jnp.float32)], axis=1)
    a_aug = jnp.dot(p, haug, preferred_element_type=jnp.float32)
    a = a_aug[:, :D_HID] / (a_aug[:, D_HID : D_HID + 1] + 1e-16)
    z = jnp.dot(a, w2_ref[...], preferred_element_type=jnp.float32) + b2_ref[...]
    zm = z - jnp.max(z, axis=1, keepdims=True)
    out_ref[...] = zm - jnp.log(jnp.sum(jnp.exp(zm), axis=1, keepdims=True))


_SC_NC, _SC_NS = 2, 16
_ROWS = 64
_NPAD = _SC_NC * _SC_NS * _ROWS
_CHUNK = 320
_N_CHUNKS = E_EDGES // _CHUNK
_NBUF = 4


def _counts_sc_body(ed_hbm, c_hbm, b0, b1, b2, b3, s0, s1, s2, s3, acc):
    bufs = (b0, b1, b2, b3)
    sems = (s0, s1, s2, s3)
    wid = lax.axis_index("s") * _SC_NC + lax.axis_index("c")
    base = wid * _ROWS
    zeros16 = jnp.zeros((16,), jnp.float32)
    ones16 = jnp.ones((16,), jnp.float32)
    lanes = lax.iota(jnp.int32, 16)

    handles = {}
    for c in range(_NBUF):
        handles[c] = pltpu.async_copy(
            ed_hbm.at[pl.ds(c * 2 * _CHUNK, 2 * _CHUNK)], bufs[c], sems[c])

    def zbody(i, carry):
        for k in range(8):
            acc[pl.ds((i * 8 + k) * 16, 16)] = zeros16
        return carry
    lax.fori_loop(0, _ROWS * N // (16 * 8), zbody, 0)

    for g in range(4):
        ln = lanes + g * 16
        col = base + ln
        plsc.addupdate_scatter(acc, [ln * N + col], ones16, mask=col < N)

    for c in range(_N_CHUNKS):
        b = bufs[c % _NBUF]
        handles[c].wait()
        if c + _NBUF < _N_CHUNKS:
            handles[c + _NBUF] = pltpu.async_copy(
                ed_hbm.at[pl.ds((c + _NBUF) * 2 * _CHUNK, 2 * _CHUNK)],
                bufs[(c + _NBUF) % _NBUF], sems[(c + _NBUF) % _NBUF])

        def vbody(i, inner):
            sv = b[pl.ds(i * 16, 16)]
            dv = b[pl.ds(_CHUNK + i * 16, 16)]
            loc = dv - base
            m = (loc >= 0) & (loc < _ROWS)
            plsc.addupdate_scatter(acc, [loc * N + sv], ones16, mask=m)
            return inner
        lax.fori_loop(0, _CHUNK // 16, vbody, 0)

    pltpu.sync_copy(acc, c_hbm.at[pl.ds(base * N, _ROWS * N)])


def _build_counts(edge_index):
    ed = jnp.concatenate(
        [edge_index[0].reshape(_N_CHUNKS, _CHUNK),
         edge_index[1].reshape(_N_CHUNKS, _CHUNK)], axis=1).reshape(-1)
    c_flat = pl.kernel(
        _counts_sc_body,
        out_type=jax.ShapeDtypeStruct((_NPAD * N,), jnp.float32),
        mesh=plsc.VectorSubcoreMesh(
            core_axis_name="c", subcore_axis_name="s",
            num_cores=_SC_NC, num_subcores=_SC_NS),
        compiler_params=pltpu.CompilerParams(needs_layout_passes=False),
        scratch_types=(
            [pltpu.VMEM((2 * _CHUNK,), jnp.int32)] * _NBUF
            + [pltpu.SemaphoreType.DMA] * _NBUF
            + [pltpu.VMEM((_ROWS * N,), jnp.float32)]
        ),
    )(ed)
    return c_flat.reshape(_NPAD, N)


def kernel(x, edge_index, W1, att_src1, att_dst1, b1, W2, att_src2, att_dst2, b2):
    f32 = jnp.float32

    c = _build_counts(edge_index)
    att2 = jnp.concatenate([att_src2, att_dst2], axis=0)
    grid1 = (N // DB,)

    h1 = pl.pallas_call(
        _layer1_pre_body,
        grid=grid1,
        in_specs=[
            pl.BlockSpec((DB, N), lambda i: (i, 0)),
            pl.BlockSpec((N, D_HID), lambda i: (0, 0)),
        ],
        out_specs=pl.BlockSpec((DB, D_HID), lambda i: (i, 0)),
        out_shape=jax.ShapeDtypeStruct((N, D_HID), f32),
    )(x, W1)

    h1a = pl.pallas_call(
        _attn1_body,
        grid=grid1,
        in_specs=[
            pl.BlockSpec((DB, N), lambda i: (i, 0)),
            pl.BlockSpec((N, D_HID), lambda i: (0, 0)),
            pl.BlockSpec((DB, D_HID), lambda i: (i, 0)),
            pl.BlockSpec((H1, F1), lambda i: (0, 0)),
            pl.BlockSpec((H1, F1), lambda i: (0, 0)),
            pl.BlockSpec((1, D_HID), lambda i: (0, 0)),
        ],
        out_specs=pl.BlockSpec((DB, D_HID), lambda i: (i, 0)),
        out_shape=jax.ShapeDtypeStruct((N, D_HID), f32),
    )(c, h1, h1, att_src1, att_dst1, b1.reshape(1, D_HID))

    out = pl.pallas_call(
        _attn2_body,
        grid=grid1,
        in_specs=[
            pl.BlockSpec((DB, N), lambda i: (i, 0)),
            pl.BlockSpec((N, D_HID), lambda i: (0, 0)),
            pl.BlockSpec((DB, D_HID), lambda i: (i, 0)),
            pl.BlockSpec((D_HID, N), lambda i: (0, 0)),
            pl.BlockSpec((2, N), lambda i: (0, 0)),
            pl.BlockSpec((1, N), lambda i: (0, 0)),
        ],
        out_specs=pl.BlockSpec((DB, N), lambda i: (i, 0)),
        out_shape=jax.ShapeDtypeStruct((N, N), f32),
    )(c, h1a, h1a, W2, att2, b2.reshape(1, N))
    return out
